# baseline (speedup 1.0000x reference)
import jax
import jax.numpy as jnp
from jax import lax
from jax.experimental import pallas as pl
from jax.experimental.pallas import tpu as pltpu
from jax.experimental.pallas import tpu_sc as plsc

NC = 2
NS = 16
NW = NC * NS
CHUNK = 128


def _make_sc_agg(n_pad, f, cpw, with_cnt):
  rows_per_tile = n_pad // NS
  nzcopies = rows_per_tile // 8
  cnt_chunks = n_pad // CHUNK
  per_tile_cnt = -(-cnt_chunks // NS)
  w_sz = 8
  nwin = cpw // w_sz

  out_type = [jax.ShapeDtypeStruct((NC, n_pad, f), jnp.float32)]
  if with_cnt:
    out_type.append(jax.ShapeDtypeStruct((NC * n_pad,), jnp.float32))

  scratch = [
      pltpu.VMEM((w_sz, CHUNK), jnp.int32),
      pltpu.VMEM((w_sz, CHUNK), jnp.int32),
      pltpu.VMEM((cpw, CHUNK), jnp.int32),
      pltpu.VMEM((CHUNK, f), jnp.float32),
      pltpu.VMEM((CHUNK, f), jnp.float32),
      pltpu.VMEM((8, f), jnp.float32),
      pltpu.VMEM_SHARED((n_pad, f), jnp.float32),
      pltpu.SemaphoreType.DMA,
      pltpu.SemaphoreType.DMA,
      pltpu.SemaphoreType.DMA,
  ]
  if with_cnt:
    scratch += [
        pltpu.VMEM((CHUNK,), jnp.float32),
        pltpu.VMEM((CHUNK,), jnp.float32),
        pltpu.VMEM_SHARED((n_pad,), jnp.float32),
    ]

  def body(h_hbm, src_hbm, dst_hbm, *refs):
    if with_cnt:
      (agg_hbm, cnt_hbm, sw_a, sw_b, dst_v, rows_a, rows_b, zb, acc,
       sem_i, sem_g, sem_s, ones_v, zc, cacc) = refs
    else:
      (agg_hbm, sw_a, sw_b, dst_v, rows_a, rows_b, zb, acc, sem_i,
       sem_g, sem_s) = refs
    c = lax.axis_index("c")
    s = lax.axis_index("s")
    w = c * NS + s

    def load_window(k, sbuf):
      pltpu.async_copy(src_hbm.at[w, pl.ds(k * w_sz, w_sz)], sbuf, sem_i)

    def wait_window(sbuf):
      pltpu.make_async_copy(src_hbm.at[0, pl.ds(0, w_sz)], sbuf,
                            sem_i).wait()

    def start_gather(sbuf, i, buf):
      pltpu.async_copy(h_hbm.at[sbuf.at[i]], buf, sem_g)

    load_window(0, sw_a)
    pltpu.sync_copy(dst_hbm.at[w], dst_v)

    z16 = jnp.zeros((16,), jnp.float32)
    for i in range(8):
      for j in range(f // 16):
        zb[i, pl.ds(j * 16, 16)] = z16
    if with_cnt:
      o16 = jnp.ones((16,), jnp.float32)
      for j in range(CHUNK // 16):
        ones_v[pl.ds(j * 16, 16)] = o16
        zc[pl.ds(j * 16, 16)] = z16

    wait_window(sw_a)
    load_window(1, sw_b)
    start_gather(sw_a, 0, rows_a)
    start_gather(sw_a, 1, rows_b)

    row0 = s * rows_per_tile

    def zero_body(i, carry):
      pltpu.async_copy(zb, acc.at[pl.ds(row0 + i * 8, 8)], sem_s)
      return carry

    lax.fori_loop(0, nzcopies, zero_body, 0)

    def zero_drain(i, carry):
      pltpu.make_async_copy(zb, acc.at[pl.ds(row0, 8)], sem_s).wait()
      return carry

    lax.fori_loop(0, nzcopies, zero_drain, 0)
    if with_cnt:
      for k in range(per_tile_cnt):
        ch = s * per_tile_cnt + k

        @pl.when(ch < cnt_chunks)
        def _():
          pltpu.sync_copy(zc, cacc.at[pl.ds(ch * CHUNK, CHUNK)])

    plsc.subcore_barrier()

    def window(k, sw, sw_other):
      base = k * w_sz
      for i in range(w_sz):
        buf = rows_a if i % 2 == 0 else rows_b
        jj = base + i
        pltpu.make_async_copy(h_hbm.at[sw.at[i]], buf, sem_g).wait()
        desc = pltpu.async_copy(buf, acc.at[dst_v.at[jj]], sem_s,
                                add=True)
        if with_cnt:
          cdesc = pltpu.async_copy(ones_v, cacc.at[dst_v.at[jj]], sem_s,
                                   add=True)
          cdesc.wait()
        desc.wait()
        if i < w_sz - 2:
          start_gather(sw, i + 2, buf)
        else:

          @pl.when(k + 1 < nwin)
          def _():
            if i == w_sz - 2:
              wait_window(sw_other)
            start_gather(sw_other, i + 2 - w_sz, buf)
      @pl.when(k + 2 < nwin)
      def _():
        load_window(k + 2, sw)

    def pair_body(p, carry):
      window(2 * p, sw_a, sw_b)
      window(2 * p + 1, sw_b, sw_a)
      return carry

    lax.fori_loop(0, nwin // 2, pair_body, 0)

    plsc.subcore_barrier()
    pltpu.sync_copy(acc.at[pl.ds(row0, rows_per_tile)],
                    agg_hbm.at[c, pl.ds(row0, rows_per_tile)])
    if with_cnt:
      for k in range(per_tile_cnt):
        ch = s * per_tile_cnt + k

        @pl.when(ch < cnt_chunks)
        def _():
          pltpu.sync_copy(cacc.at[pl.ds(ch * CHUNK, CHUNK)], zc)
          pltpu.sync_copy(zc, cnt_hbm.at[pl.ds(c * n_pad + ch * CHUNK,
                                               CHUNK)])

  mesh = plsc.VectorSubcoreMesh(core_axis_name="c", subcore_axis_name="s",
                                num_cores=NC, num_subcores=NS)
  params = None
  if f % 128 != 0:
    params = pltpu.CompilerParams(use_tc_tiling_on_sc=False)
  return pl.kernel(body, out_type=tuple(out_type) if with_cnt else out_type[0],
                   mesh=mesh, scratch_types=scratch,
                   compiler_params=params)


def _tc_first(agg, cnt3, x, wl0, wr0, b0, wl1, wr1, b1):
  n = x.shape[0]
  h = wl1.shape[1]

  def body(p_ref, c_ref, x_ref, wl0_ref, wr0_ref, b0_ref, wl1_ref,
           wr1_ref, b1_ref, y_ref, z_ref, inv_ref):
    cnt = c_ref[0, 0:n, :] + c_ref[1, 0:n, :]
    inv = 1.0 / jnp.maximum(cnt, 1.0)
    inv_ref[...] = inv
    mean = (p_ref[0, 0:n, :] + p_ref[1, 0:n, :]) * inv
    hv = (jnp.dot(mean, wl0_ref[...], preferred_element_type=jnp.float32)
          + jnp.dot(x_ref[...], wr0_ref[...],
                    preferred_element_type=jnp.float32) + b0_ref[...])
    hv = jnp.maximum(hv, 0.0)
    y_ref[...] = jnp.dot(hv, wl1_ref[...],
                         preferred_element_type=jnp.float32)
    z_ref[...] = (jnp.dot(hv, wr1_ref[...],
                          preferred_element_type=jnp.float32)
                  + b1_ref[...])

  return pl.pallas_call(
      body,
      out_shape=(jax.ShapeDtypeStruct((n, h), jnp.float32),
                 jax.ShapeDtypeStruct((n, h), jnp.float32),
                 jax.ShapeDtypeStruct((n, 1), jnp.float32)),
  )(agg, cnt3, x, wl0, wr0, b0, wl1, wr1, b1)


def _tc_mid(agg, inv, z, wl, wr, b):
  n = z.shape[0]
  h = wl.shape[1]

  def body(p_ref, i_ref, z_ref, wl_ref, wr_ref, b_ref, y_ref, zn_ref):
    agg = p_ref[0, 0:n, :] + p_ref[1, 0:n, :]
    hv = jnp.maximum(agg * i_ref[...] + z_ref[...], 0.0)
    y_ref[...] = jnp.dot(hv, wl_ref[...],
                         preferred_element_type=jnp.float32)
    zn_ref[...] = (jnp.dot(hv, wr_ref[...],
                           preferred_element_type=jnp.float32)
                   + b_ref[...])

  return pl.pallas_call(
      body,
      out_shape=(jax.ShapeDtypeStruct((n, h), jnp.float32),
                 jax.ShapeDtypeStruct((n, h), jnp.float32)),
  )(agg, inv, z, wl, wr, b)


def _tc_final(agg, inv, z):
  n, c = z.shape

  def body(p_ref, i_ref, z_ref, o_ref):
    agg = p_ref[0, 0:n, :] + p_ref[1, 0:n, :]
    out = agg * i_ref[...] + z_ref[...]
    nrm = jnp.sqrt(jnp.sum(out * out, axis=1, keepdims=True))
    out = out / jnp.maximum(nrm, 1e-12)
    m = jnp.max(out, axis=1, keepdims=True)
    o_ref[...] = out - (m + jnp.log(jnp.sum(jnp.exp(out - m), axis=1,
                                            keepdims=True)))

  return pl.pallas_call(
      body,
      out_shape=jax.ShapeDtypeStruct((n, c), jnp.float32),
  )(agg, inv, z)


def kernel(x, edge_index, Wl0, Wr0, b0, Wl1, Wr1, b1, Wl2, Wr2, b2):
  n, f = x.shape
  e = edge_index.shape[1]

  cpw = -(-e // (NW * CHUNK))
  cpw = -(-cpw // 16) * 16
  e_pad = NW * cpw * CHUNK
  n_pad = -(-n // CHUNK) * CHUNK
  if e_pad > e and n_pad == n:
    n_pad += CHUNK

  src = edge_index[0]
  dst = edge_index[1]
  pad = e_pad - e
  if pad:
    pidx = jnp.arange(pad, dtype=jnp.int32)
    src = jnp.concatenate([src, pidx % n])
    dst = jnp.concatenate([dst, n + pidx % (n_pad - n)])
  src = src.reshape(NW, cpw, CHUNK)
  dst = dst.reshape(NW, cpw, CHUNK)

  h = Wl0.shape[1]
  c_out = Wl2.shape[1]
  sc_first = _make_sc_agg(n_pad, h, cpw, True)
  sc_mid = _make_sc_agg(n_pad, h, cpw, False)
  sc_last = _make_sc_agg(n_pad, c_out, cpw, False)

  agg0, cnt = sc_first(x, src, dst)
  cnt3 = cnt.reshape(NC, n_pad, 1)
  y1, z1, inv = _tc_first(agg0, cnt3, x, Wl0, Wr0, b0.reshape(1, -1),
                          Wl1, Wr1, b1.reshape(1, -1))
  agg1 = sc_mid(y1, src, dst)
  y2, z2 = _tc_mid(agg1, inv, z1, Wl2, Wr2, b2.reshape(1, -1))
  agg2 = sc_last(y2, src, dst)
  return _tc_final(agg2, inv, z2)


# speedup vs baseline: 1.0007x; 1.0007x over previous
import jax
import jax.numpy as jnp
from jax import lax
from jax.experimental import pallas as pl
from jax.experimental.pallas import tpu as pltpu
from jax.experimental.pallas import tpu_sc as plsc

NC = 2
NS = 16
NW = NC * NS
CHUNK = 128


def _make_sc_agg(n_pad, f, cpw, with_cnt):
  rows_per_tile = n_pad // NS
  nzcopies = rows_per_tile // 8
  cnt_chunks = n_pad // CHUNK
  per_tile_cnt = -(-cnt_chunks // NS)
  w_sz = 8
  nwin = cpw // w_sz

  out_type = [jax.ShapeDtypeStruct((NC, n_pad, f), jnp.float32)]
  if with_cnt:
    out_type.append(jax.ShapeDtypeStruct((NC * n_pad,), jnp.float32))

  scratch = [
      pltpu.VMEM((w_sz, CHUNK), jnp.int32),
      pltpu.VMEM((w_sz, CHUNK), jnp.int32),
      pltpu.VMEM((cpw, CHUNK), jnp.int32),
      pltpu.VMEM((CHUNK, f), jnp.float32),
      pltpu.VMEM((CHUNK, f), jnp.float32),
      pltpu.VMEM((8, f), jnp.float32),
      pltpu.VMEM_SHARED((n_pad, f), jnp.float32),
      pltpu.SemaphoreType.DMA,
      pltpu.SemaphoreType.DMA,
      pltpu.SemaphoreType.DMA,
  ]
  if with_cnt:
    scratch += [
        pltpu.VMEM((CHUNK,), jnp.float32),
        pltpu.VMEM((CHUNK,), jnp.float32),
        pltpu.VMEM_SHARED((n_pad,), jnp.float32),
    ]

  def body(h_hbm, src_hbm, dst_hbm, *refs):
    if with_cnt:
      (agg_hbm, cnt_hbm, sw_a, sw_b, dst_v, rows_a, rows_b, zb, acc,
       sem_i, sem_g, sem_s, ones_v, zc, cacc) = refs
    else:
      (agg_hbm, sw_a, sw_b, dst_v, rows_a, rows_b, zb, acc, sem_i,
       sem_g, sem_s) = refs
    c = lax.axis_index("c")
    s = lax.axis_index("s")
    w = c * NS + s

    def load_window(k, sbuf):
      pltpu.async_copy(src_hbm.at[w, pl.ds(k * w_sz, w_sz)], sbuf, sem_i)

    def wait_window(sbuf):
      pltpu.make_async_copy(src_hbm.at[0, pl.ds(0, w_sz)], sbuf,
                            sem_i).wait()

    def start_gather(sbuf, i, buf):
      pltpu.async_copy(h_hbm.at[sbuf.at[i]], buf, sem_g)

    load_window(0, sw_a)
    pltpu.sync_copy(dst_hbm.at[w], dst_v)

    z16 = jnp.zeros((16,), jnp.float32)
    for i in range(8):
      for j in range(f // 16):
        zb[i, pl.ds(j * 16, 16)] = z16
    if with_cnt:
      o16 = jnp.ones((16,), jnp.float32)
      for j in range(CHUNK // 16):
        ones_v[pl.ds(j * 16, 16)] = o16
        zc[pl.ds(j * 16, 16)] = z16

    wait_window(sw_a)
    load_window(1, sw_b)
    start_gather(sw_a, 0, rows_a)
    start_gather(sw_a, 1, rows_b)

    row0 = s * rows_per_tile

    def zero_body(i, carry):
      pltpu.async_copy(zb, acc.at[pl.ds(row0 + i * 8, 8)], sem_s)
      return carry

    lax.fori_loop(0, nzcopies, zero_body, 0)

    def zero_drain(i, carry):
      pltpu.make_async_copy(zb, acc.at[pl.ds(row0, 8)], sem_s).wait()
      return carry

    lax.fori_loop(0, nzcopies, zero_drain, 0)
    if with_cnt:
      for k in range(per_tile_cnt):
        ch = s * per_tile_cnt + k

        @pl.when(ch < cnt_chunks)
        def _():
          pltpu.sync_copy(zc, cacc.at[pl.ds(ch * CHUNK, CHUNK)])

    plsc.subcore_barrier()

    def window(k, sw, sw_other):
      base = k * w_sz
      for i in range(w_sz):
        buf = rows_a if i % 2 == 0 else rows_b
        jj = base + i
        pltpu.make_async_copy(h_hbm.at[sw.at[i]], buf, sem_g).wait()
        desc = pltpu.async_copy(buf, acc.at[dst_v.at[jj]], sem_s,
                                add=True)
        if with_cnt:
          cdesc = pltpu.async_copy(ones_v, cacc.at[dst_v.at[jj]], sem_s,
                                   add=True)
          cdesc.wait()
        desc.wait()
        if i < w_sz - 2:
          start_gather(sw, i + 2, buf)
        else:

          @pl.when(k + 1 < nwin)
          def _():
            if i == w_sz - 2:
              wait_window(sw_other)
            start_gather(sw_other, i + 2 - w_sz, buf)
      @pl.when(k + 2 < nwin)
      def _():
        load_window(k + 2, sw)

    def pair_body(p, carry):
      window(2 * p, sw_a, sw_b)
      window(2 * p + 1, sw_b, sw_a)
      return carry

    lax.fori_loop(0, nwin // 2, pair_body, 0)

    plsc.subcore_barrier()
    pltpu.sync_copy(acc.at[pl.ds(row0, rows_per_tile)],
                    agg_hbm.at[c, pl.ds(row0, rows_per_tile)])
    if with_cnt:
      for k in range(per_tile_cnt):
        ch = s * per_tile_cnt + k

        @pl.when(ch < cnt_chunks)
        def _():
          pltpu.sync_copy(cacc.at[pl.ds(ch * CHUNK, CHUNK)], zc)
          pltpu.sync_copy(zc, cnt_hbm.at[pl.ds(c * n_pad + ch * CHUNK,
                                               CHUNK)])

  mesh = plsc.VectorSubcoreMesh(core_axis_name="c", subcore_axis_name="s",
                                num_cores=NC, num_subcores=NS)
  params = None
  if f % 128 != 0:
    params = pltpu.CompilerParams(use_tc_tiling_on_sc=False)
  return pl.kernel(body, out_type=tuple(out_type) if with_cnt else out_type[0],
                   mesh=mesh, scratch_types=scratch,
                   compiler_params=params)


def _tc_first(agg, cnt3, x, wl0, wr0, b0, wl1, wr1, b1):
  n = x.shape[0]
  h = wl1.shape[1]

  def body(p_ref, c_ref, x_ref, wl0_ref, wr0_ref, b0_ref, wl1_ref,
           wr1_ref, b1_ref, y_ref, h_ref, inv_ref):
    cnt = c_ref[0, 0:n, :] + c_ref[1, 0:n, :]
    inv = 1.0 / jnp.maximum(cnt, 1.0)
    inv_ref[...] = inv
    mean = (p_ref[0, 0:n, :] + p_ref[1, 0:n, :]) * inv
    hv = (jnp.dot(mean, wl0_ref[...], preferred_element_type=jnp.float32)
          + jnp.dot(x_ref[...], wr0_ref[...],
                    preferred_element_type=jnp.float32) + b0_ref[...])
    hv = jnp.maximum(hv, 0.0)
    h_ref[...] = hv
    y_ref[...] = jnp.dot(hv, wl1_ref[...],
                         preferred_element_type=jnp.float32)

  return pl.pallas_call(
      body,
      out_shape=(jax.ShapeDtypeStruct((n, h), jnp.float32),
                 jax.ShapeDtypeStruct((n, h), jnp.float32),
                 jax.ShapeDtypeStruct((n, 1), jnp.float32)),
  )(agg, cnt3, x, wl0, wr0, b0, wl1, wr1, b1)


def _tc_z(hv, wr, b):
  n = hv.shape[0]
  c = wr.shape[1]

  def body(h_ref, wr_ref, b_ref, z_ref):
    z_ref[...] = (jnp.dot(h_ref[...], wr_ref[...],
                          preferred_element_type=jnp.float32)
                  + b_ref[...])

  return pl.pallas_call(
      body,
      out_shape=jax.ShapeDtypeStruct((n, c), jnp.float32),
  )(hv, wr, b)


def _tc_mid(agg, inv, z, wl):
  n, h = z.shape

  def body(p_ref, i_ref, z_ref, wl_ref, y_ref, h_ref):
    agg = p_ref[0, 0:n, :] + p_ref[1, 0:n, :]
    hv = jnp.maximum(agg * i_ref[...] + z_ref[...], 0.0)
    h_ref[...] = hv
    y_ref[...] = jnp.dot(hv, wl_ref[...],
                         preferred_element_type=jnp.float32)

  return pl.pallas_call(
      body,
      out_shape=(jax.ShapeDtypeStruct((n, wl.shape[1]), jnp.float32),
                 jax.ShapeDtypeStruct((n, h), jnp.float32)),
  )(agg, inv, z, wl)


def _tc_final(agg, inv, z):
  n, c = z.shape

  def body(p_ref, i_ref, z_ref, o_ref):
    agg = p_ref[0, 0:n, :] + p_ref[1, 0:n, :]
    out = agg * i_ref[...] + z_ref[...]
    nrm = jnp.sqrt(jnp.sum(out * out, axis=1, keepdims=True))
    out = out / jnp.maximum(nrm, 1e-12)
    m = jnp.max(out, axis=1, keepdims=True)
    o_ref[...] = out - (m + jnp.log(jnp.sum(jnp.exp(out - m), axis=1,
                                            keepdims=True)))

  return pl.pallas_call(
      body,
      out_shape=jax.ShapeDtypeStruct((n, c), jnp.float32),
  )(agg, inv, z)


def kernel(x, edge_index, Wl0, Wr0, b0, Wl1, Wr1, b1, Wl2, Wr2, b2):
  n, f = x.shape
  e = edge_index.shape[1]

  cpw = -(-e // (NW * CHUNK))
  cpw = -(-cpw // 16) * 16
  e_pad = NW * cpw * CHUNK
  n_pad = -(-n // CHUNK) * CHUNK
  if e_pad > e and n_pad == n:
    n_pad += CHUNK

  src = edge_index[0]
  dst = edge_index[1]
  pad = e_pad - e
  if pad:
    pidx = jnp.arange(pad, dtype=jnp.int32)
    src = jnp.concatenate([src, pidx % n])
    dst = jnp.concatenate([dst, n + pidx % (n_pad - n)])
  src = src.reshape(NW, cpw, CHUNK)
  dst = dst.reshape(NW, cpw, CHUNK)

  h = Wl0.shape[1]
  c_out = Wl2.shape[1]
  sc_first = _make_sc_agg(n_pad, h, cpw, True)
  sc_mid = _make_sc_agg(n_pad, h, cpw, False)
  sc_last = _make_sc_agg(n_pad, c_out, cpw, False)

  agg0, cnt = sc_first(x, src, dst)
  cnt3 = cnt.reshape(NC, n_pad, 1)
  y1, h1, inv = _tc_first(agg0, cnt3, x, Wl0, Wr0, b0.reshape(1, -1),
                          Wl1, Wr1, b1.reshape(1, -1))
  agg1 = sc_mid(y1, src, dst)
  z1 = _tc_z(h1, Wr1, b1.reshape(1, -1))
  y2, h2 = _tc_mid(agg1, inv, z1, Wl2)
  agg2 = sc_last(y2, src, dst)
  z2 = _tc_z(h2, Wr2, b2.reshape(1, -1))
  return _tc_final(agg2, inv, z2)


# speedup vs baseline: 1.0093x; 1.0086x over previous
import jax
import jax.numpy as jnp
from jax import lax
from jax.experimental import pallas as pl
from jax.experimental.pallas import tpu as pltpu
from jax.experimental.pallas import tpu_sc as plsc

NC = 2
NS = 16
NW = NC * NS
CHUNK = 128


def _make_sc_agg(n_pad, f, cpw, with_cnt):
  rows_per_tile = n_pad // NS
  nzcopies = rows_per_tile // 8
  cnt_chunks = n_pad // CHUNK
  per_tile_cnt = -(-cnt_chunks // NS)
  w_sz = 8
  nwin = cpw // w_sz

  out_type = [jax.ShapeDtypeStruct((NC, n_pad, f), jnp.float32)]
  if with_cnt:
    out_type.append(jax.ShapeDtypeStruct((NC * n_pad,), jnp.float32))

  scratch = [
      pltpu.VMEM((w_sz, CHUNK), jnp.int32),
      pltpu.VMEM((w_sz, CHUNK), jnp.int32),
      pltpu.VMEM((cpw, CHUNK), jnp.int32),
      pltpu.VMEM((CHUNK, f), jnp.float32),
      pltpu.VMEM((CHUNK, f), jnp.float32),
      pltpu.VMEM((8, f), jnp.float32),
      pltpu.VMEM_SHARED((n_pad, f), jnp.float32),
      pltpu.SemaphoreType.DMA,
      pltpu.SemaphoreType.DMA,
      pltpu.SemaphoreType.DMA,
  ]
  if with_cnt:
    scratch += [
        pltpu.VMEM((CHUNK,), jnp.float32),
        pltpu.VMEM((CHUNK,), jnp.float32),
        pltpu.VMEM_SHARED((n_pad,), jnp.float32),
    ]

  def body(h_hbm, src_hbm, dst_hbm, *refs):
    if with_cnt:
      (agg_hbm, cnt_hbm, sw_a, sw_b, dst_v, rows_a, rows_b, zb, acc,
       sem_i, sem_g, sem_s, ones_v, zc, cacc) = refs
    else:
      (agg_hbm, sw_a, sw_b, dst_v, rows_a, rows_b, zb, acc, sem_i,
       sem_g, sem_s) = refs
    c = lax.axis_index("c")
    s = lax.axis_index("s")
    w = c * NS + s

    def load_window(k, sbuf):
      pltpu.async_copy(src_hbm.at[w, pl.ds(k * w_sz, w_sz)], sbuf, sem_i)

    def wait_window(sbuf):
      pltpu.make_async_copy(src_hbm.at[0, pl.ds(0, w_sz)], sbuf,
                            sem_i).wait()

    def start_gather(sbuf, i, buf):
      pltpu.async_copy(h_hbm.at[sbuf.at[i]], buf, sem_g)

    load_window(0, sw_a)
    pltpu.sync_copy(dst_hbm.at[w], dst_v)

    z16 = jnp.zeros((16,), jnp.float32)
    for i in range(8):
      for j in range(f // 16):
        zb[i, pl.ds(j * 16, 16)] = z16
    if with_cnt:
      o16 = jnp.ones((16,), jnp.float32)
      for j in range(CHUNK // 16):
        ones_v[pl.ds(j * 16, 16)] = o16
        zc[pl.ds(j * 16, 16)] = z16

    wait_window(sw_a)
    load_window(1, sw_b)
    start_gather(sw_a, 0, rows_a)
    start_gather(sw_a, 1, rows_b)

    row0 = s * rows_per_tile

    def zero_body(i, carry):
      pltpu.async_copy(zb, acc.at[pl.ds(row0 + i * 8, 8)], sem_s)
      return carry

    lax.fori_loop(0, nzcopies, zero_body, 0)

    def zero_drain(i, carry):
      pltpu.make_async_copy(zb, acc.at[pl.ds(row0, 8)], sem_s).wait()
      return carry

    lax.fori_loop(0, nzcopies, zero_drain, 0)
    if with_cnt:
      for k in range(per_tile_cnt):
        ch = s * per_tile_cnt + k

        @pl.when(ch < cnt_chunks)
        def _():
          pltpu.sync_copy(zc, cacc.at[pl.ds(ch * CHUNK, CHUNK)])

    plsc.subcore_barrier()

    def window(k, sw, sw_other):
      base = k * w_sz
      for i in range(w_sz):
        buf = rows_a if i % 2 == 0 else rows_b
        jj = base + i
        pltpu.make_async_copy(h_hbm.at[sw.at[i]], buf, sem_g).wait()
        desc = pltpu.async_copy(buf, acc.at[dst_v.at[jj]], sem_s,
                                add=True)
        if with_cnt:
          cdesc = pltpu.async_copy(ones_v, cacc.at[dst_v.at[jj]], sem_s,
                                   add=True)
          cdesc.wait()
        desc.wait()
        if i < w_sz - 2:
          start_gather(sw, i + 2, buf)
        else:

          @pl.when(k + 1 < nwin)
          def _():
            if i == w_sz - 2:
              wait_window(sw_other)
            start_gather(sw_other, i + 2 - w_sz, buf)
      @pl.when(k + 2 < nwin)
      def _():
        load_window(k + 2, sw)

    def pair_body(p, carry):
      window(2 * p, sw_a, sw_b)
      window(2 * p + 1, sw_b, sw_a)
      return carry

    lax.fori_loop(0, nwin // 2, pair_body, 0)

    plsc.subcore_barrier()
    pltpu.sync_copy(acc.at[pl.ds(row0, rows_per_tile)],
                    agg_hbm.at[c, pl.ds(row0, rows_per_tile)])
    if with_cnt:
      for k in range(per_tile_cnt):
        ch = s * per_tile_cnt + k

        @pl.when(ch < cnt_chunks)
        def _():
          pltpu.sync_copy(cacc.at[pl.ds(ch * CHUNK, CHUNK)], zc)
          pltpu.sync_copy(zc, cnt_hbm.at[pl.ds(c * n_pad + ch * CHUNK,
                                               CHUNK)])

  mesh = plsc.VectorSubcoreMesh(core_axis_name="c", subcore_axis_name="s",
                                num_cores=NC, num_subcores=NS)
  params = None
  if f % 128 != 0:
    params = pltpu.CompilerParams(use_tc_tiling_on_sc=False)
  return pl.kernel(body, out_type=tuple(out_type) if with_cnt else out_type[0],
                   mesh=mesh, scratch_types=scratch,
                   compiler_params=params)


BN = 2000


def _row_spec(width):
  return pl.BlockSpec((BN, width), lambda i: (i, 0))


def _part_spec(width):
  return pl.BlockSpec((NC, BN, width), lambda i: (0, i, 0))


def _full_spec(a):
  return pl.BlockSpec(a.shape, lambda i: (0,) * a.ndim)


def _tc_first(agg, cnt3, x, wl0, wr0, b0, wl1, wr1, b1):
  n = x.shape[0]
  h = wl1.shape[1]

  def body(p_ref, c_ref, x_ref, wl0_ref, wr0_ref, b0_ref, wl1_ref,
           y_ref, h_ref, inv_ref):
    cnt = c_ref[0] + c_ref[1]
    inv = 1.0 / jnp.maximum(cnt, 1.0)
    inv_ref[...] = inv
    mean = (p_ref[0] + p_ref[1]) * inv
    hv = (jnp.dot(mean, wl0_ref[...], preferred_element_type=jnp.float32)
          + jnp.dot(x_ref[...], wr0_ref[...],
                    preferred_element_type=jnp.float32) + b0_ref[...])
    hv = jnp.maximum(hv, 0.0)
    h_ref[...] = hv
    y_ref[...] = jnp.dot(hv, wl1_ref[...],
                         preferred_element_type=jnp.float32)

  f = x.shape[1]
  return pl.pallas_call(
      body,
      grid=(n // BN,),
      in_specs=[_part_spec(f), _part_spec(1), _row_spec(f),
                _full_spec(wl0), _full_spec(wr0), _full_spec(b0),
                _full_spec(wl1)],
      out_specs=(_row_spec(h), _row_spec(h), _row_spec(1)),
      out_shape=(jax.ShapeDtypeStruct((n, h), jnp.float32),
                 jax.ShapeDtypeStruct((n, h), jnp.float32),
                 jax.ShapeDtypeStruct((n, 1), jnp.float32)),
  )(agg, cnt3, x, wl0, wr0, b0, wl1)


def _tc_z(hv, wr, b):
  n, f = hv.shape
  c = wr.shape[1]

  def body(h_ref, wr_ref, b_ref, z_ref):
    z_ref[...] = (jnp.dot(h_ref[...], wr_ref[...],
                          preferred_element_type=jnp.float32)
                  + b_ref[...])

  return pl.pallas_call(
      body,
      grid=(n // BN,),
      in_specs=[_row_spec(f), _full_spec(wr), _full_spec(b)],
      out_specs=_row_spec(c),
      out_shape=jax.ShapeDtypeStruct((n, c), jnp.float32),
  )(hv, wr, b)


def _tc_mid(agg, inv, z, wl):
  n, h = z.shape
  c = wl.shape[1]

  def body(p_ref, i_ref, z_ref, wl_ref, y_ref, h_ref):
    hv = jnp.maximum((p_ref[0] + p_ref[1]) * i_ref[...] + z_ref[...],
                     0.0)
    h_ref[...] = hv
    y_ref[...] = jnp.dot(hv, wl_ref[...],
                         preferred_element_type=jnp.float32)

  return pl.pallas_call(
      body,
      grid=(n // BN,),
      in_specs=[_part_spec(h), _row_spec(1), _row_spec(h),
                _full_spec(wl)],
      out_specs=(_row_spec(c), _row_spec(h)),
      out_shape=(jax.ShapeDtypeStruct((n, c), jnp.float32),
                 jax.ShapeDtypeStruct((n, h), jnp.float32)),
  )(agg, inv, z, wl)


def _tc_final(agg, inv, z):
  n, c = z.shape

  def body(p_ref, i_ref, z_ref, o_ref):
    out = (p_ref[0] + p_ref[1]) * i_ref[...] + z_ref[...]
    nrm = jnp.sqrt(jnp.sum(out * out, axis=1, keepdims=True))
    out = out / jnp.maximum(nrm, 1e-12)
    m = jnp.max(out, axis=1, keepdims=True)
    o_ref[...] = out - (m + jnp.log(jnp.sum(jnp.exp(out - m), axis=1,
                                            keepdims=True)))

  return pl.pallas_call(
      body,
      grid=(n // BN,),
      in_specs=[_part_spec(c), _row_spec(1), _row_spec(c)],
      out_specs=_row_spec(c),
      out_shape=jax.ShapeDtypeStruct((n, c), jnp.float32),
  )(agg, inv, z)


def kernel(x, edge_index, Wl0, Wr0, b0, Wl1, Wr1, b1, Wl2, Wr2, b2):
  n, f = x.shape
  e = edge_index.shape[1]

  cpw = -(-e // (NW * CHUNK))
  cpw = -(-cpw // 16) * 16
  e_pad = NW * cpw * CHUNK
  n_pad = -(-n // CHUNK) * CHUNK
  if e_pad > e and n_pad == n:
    n_pad += CHUNK

  src = edge_index[0]
  dst = edge_index[1]
  pad = e_pad - e
  if pad:
    pidx = jnp.arange(pad, dtype=jnp.int32)
    src = jnp.concatenate([src, pidx % n])
    dst = jnp.concatenate([dst, n + pidx % (n_pad - n)])
  src = src.reshape(NW, cpw, CHUNK)
  dst = dst.reshape(NW, cpw, CHUNK)

  h = Wl0.shape[1]
  c_out = Wl2.shape[1]
  sc_first = _make_sc_agg(n_pad, h, cpw, True)
  sc_mid = _make_sc_agg(n_pad, h, cpw, False)
  sc_last = _make_sc_agg(n_pad, c_out, cpw, False)

  agg0, cnt = sc_first(x, src, dst)
  cnt3 = cnt.reshape(NC, n_pad, 1)
  y1, h1, inv = _tc_first(agg0, cnt3, x, Wl0, Wr0, b0.reshape(1, -1),
                          Wl1, Wr1, b1.reshape(1, -1))
  agg1 = sc_mid(y1, src, dst)
  z1 = _tc_z(h1, Wr1, b1.reshape(1, -1))
  y2, h2 = _tc_mid(agg1, inv, z1, Wl2)
  agg2 = sc_last(y2, src, dst)
  z2 = _tc_z(h2, Wr2, b2.reshape(1, -1))
  return _tc_final(agg2, inv, z2)


# speedup vs baseline: 1.0553x; 1.0456x over previous
import jax
import jax.numpy as jnp
import numpy as np
from jax import lax
from jax.experimental import pallas as pl
from jax.experimental.pallas import tpu as pltpu
from jax.experimental.pallas import tpu_sc as plsc

NC = 2
NS = 16
NW = NC * NS
CHUNK = 128


def _make_sc_agg(n_pad, f, cpw, with_cnt):
  rows_per_tile = n_pad // NS
  nzcopies = rows_per_tile // 8
  cnt_chunks = n_pad // CHUNK
  per_tile_cnt = -(-cnt_chunks // NS)
  w_sz = 8
  nwin = cpw // w_sz

  out_type = [jax.ShapeDtypeStruct((NC, n_pad, f), jnp.float32)]
  if with_cnt:
    out_type.append(jax.ShapeDtypeStruct((NC * n_pad,), jnp.float32))

  scratch = [
      pltpu.VMEM((w_sz, CHUNK), jnp.int32),
      pltpu.VMEM((w_sz, CHUNK), jnp.int32),
      pltpu.VMEM((cpw, CHUNK), jnp.int32),
      pltpu.VMEM((CHUNK, f), jnp.float32),
      pltpu.VMEM((CHUNK, f), jnp.float32),
      pltpu.VMEM((8, f), jnp.float32),
      pltpu.VMEM_SHARED((n_pad, f), jnp.float32),
      pltpu.SemaphoreType.DMA,
      pltpu.SemaphoreType.DMA,
      pltpu.SemaphoreType.DMA,
  ]
  if with_cnt:
    scratch += [
        pltpu.VMEM((CHUNK,), jnp.float32),
        pltpu.VMEM((CHUNK,), jnp.float32),
        pltpu.VMEM_SHARED((n_pad,), jnp.float32),
    ]

  def body(h_hbm, src_hbm, dst_hbm, *refs):
    if with_cnt:
      (agg_hbm, cnt_hbm, sw_a, sw_b, dst_v, rows_a, rows_b, zb, acc,
       sem_i, sem_g, sem_s, ones_v, zc, cacc) = refs
    else:
      (agg_hbm, sw_a, sw_b, dst_v, rows_a, rows_b, zb, acc, sem_i,
       sem_g, sem_s) = refs
    c = lax.axis_index("c")
    s = lax.axis_index("s")
    w = c * NS + s

    def load_window(k, sbuf):
      pltpu.async_copy(src_hbm.at[w, pl.ds(k * w_sz, w_sz)], sbuf, sem_i)

    def wait_window(sbuf):
      pltpu.make_async_copy(src_hbm.at[0, pl.ds(0, w_sz)], sbuf,
                            sem_i).wait()

    def start_gather(sbuf, i, buf):
      pltpu.async_copy(h_hbm.at[sbuf.at[i]], buf, sem_g)

    load_window(0, sw_a)
    pltpu.sync_copy(dst_hbm.at[w], dst_v)

    z16 = jnp.zeros((16,), jnp.float32)
    for i in range(8):
      for j in range(f // 16):
        zb[i, pl.ds(j * 16, 16)] = z16
    if with_cnt:
      o16 = jnp.ones((16,), jnp.float32)
      for j in range(CHUNK // 16):
        ones_v[pl.ds(j * 16, 16)] = o16
        zc[pl.ds(j * 16, 16)] = z16

    wait_window(sw_a)
    load_window(1, sw_b)
    start_gather(sw_a, 0, rows_a)
    start_gather(sw_a, 1, rows_b)

    row0 = s * rows_per_tile

    def zero_body(i, carry):
      pltpu.async_copy(zb, acc.at[pl.ds(row0 + i * 8, 8)], sem_s)
      return carry

    lax.fori_loop(0, nzcopies, zero_body, 0)

    def zero_drain(i, carry):
      pltpu.make_async_copy(zb, acc.at[pl.ds(row0, 8)], sem_s).wait()
      return carry

    lax.fori_loop(0, nzcopies, zero_drain, 0)
    if with_cnt:
      for k in range(per_tile_cnt):
        ch = s * per_tile_cnt + k

        @pl.when(ch < cnt_chunks)
        def _():
          pltpu.sync_copy(zc, cacc.at[pl.ds(ch * CHUNK, CHUNK)])

    plsc.subcore_barrier()

    def window(k, sw, sw_other):
      base = k * w_sz
      for i in range(w_sz):
        buf = rows_a if i % 2 == 0 else rows_b
        jj = base + i
        pltpu.make_async_copy(h_hbm.at[sw.at[i]], buf, sem_g).wait()
        desc = pltpu.async_copy(buf, acc.at[dst_v.at[jj]], sem_s,
                                add=True)
        if with_cnt:
          cdesc = pltpu.async_copy(ones_v, cacc.at[dst_v.at[jj]], sem_s,
                                   add=True)
          cdesc.wait()
        desc.wait()
        if i < w_sz - 2:
          start_gather(sw, i + 2, buf)
        else:

          @pl.when(k + 1 < nwin)
          def _():
            if i == w_sz - 2:
              wait_window(sw_other)
            start_gather(sw_other, i + 2 - w_sz, buf)
      @pl.when(k + 2 < nwin)
      def _():
        load_window(k + 2, sw)

    def pair_body(p, carry):
      window(2 * p, sw_a, sw_b)
      window(2 * p + 1, sw_b, sw_a)
      return carry

    lax.fori_loop(0, nwin // 2, pair_body, 0)

    plsc.subcore_barrier()
    pltpu.sync_copy(acc.at[pl.ds(row0, rows_per_tile)],
                    agg_hbm.at[c, pl.ds(row0, rows_per_tile)])
    if with_cnt:
      for k in range(per_tile_cnt):
        ch = s * per_tile_cnt + k

        @pl.when(ch < cnt_chunks)
        def _():
          pltpu.sync_copy(cacc.at[pl.ds(ch * CHUNK, CHUNK)], zc)
          pltpu.sync_copy(zc, cnt_hbm.at[pl.ds(c * n_pad + ch * CHUNK,
                                               CHUNK)])

  mesh = plsc.VectorSubcoreMesh(core_axis_name="c", subcore_axis_name="s",
                                num_cores=NC, num_subcores=NS)
  params = None
  if f % 128 != 0:
    params = pltpu.CompilerParams(use_tc_tiling_on_sc=False)
  return pl.kernel(body, out_type=tuple(out_type) if with_cnt else out_type[0],
                   mesh=mesh, scratch_types=scratch,
                   compiler_params=params)


BN = 2048


def _row_spec(width):
  return pl.BlockSpec((BN, width), lambda i: (i, 0))


def _vec_spec():
  return pl.BlockSpec((BN,), lambda i: (i,))


def _part_spec(width):
  return pl.BlockSpec((NC, BN, width), lambda i: (0, i, 0))


def _full_spec(a):
  return pl.BlockSpec(a.shape, lambda i: (0,) * a.ndim)


def _grid(n):
  return (-(-n // BN),)


def _tc_first(agg, c0, c1, x, wl0, wr0, b0, wl1):
  n = x.shape[0]
  h = wl1.shape[1]

  def body(p_ref, c0_ref, c1_ref, x_ref, wl0_ref, wr0_ref, b0_ref,
           wl1_ref, y_ref, h_ref, inv_ref):
    inv = 1.0 / jnp.maximum(c0_ref[...] + c1_ref[...], 1.0)
    inv_ref[...] = inv
    mean = (p_ref[0] + p_ref[1]) * inv.reshape(BN, 1)
    hv = (jnp.dot(mean, wl0_ref[...], preferred_element_type=jnp.float32)
          + jnp.dot(x_ref[...], wr0_ref[...],
                    preferred_element_type=jnp.float32) + b0_ref[...])
    hv = jnp.maximum(hv, 0.0)
    h_ref[...] = hv
    y_ref[...] = jnp.dot(hv, wl1_ref[...],
                         preferred_element_type=jnp.float32)

  f = x.shape[1]
  return pl.pallas_call(
      body,
      grid=_grid(n),
      in_specs=[_part_spec(f), _vec_spec(), _vec_spec(), _row_spec(f),
                _full_spec(wl0), _full_spec(wr0), _full_spec(b0),
                _full_spec(wl1)],
      out_specs=(_row_spec(h), _row_spec(h), _vec_spec()),
      out_shape=(jax.ShapeDtypeStruct((n, h), jnp.float32),
                 jax.ShapeDtypeStruct((n, h), jnp.float32),
                 jax.ShapeDtypeStruct((n,), jnp.float32)),
  )(agg, c0, c1, x, wl0, wr0, b0, wl1)


def _tc_z(hv, wr, b):
  n, f = hv.shape
  c = wr.shape[1]

  def body(h_ref, wr_ref, b_ref, z_ref):
    z_ref[...] = (jnp.dot(h_ref[...], wr_ref[...],
                          preferred_element_type=jnp.float32)
                  + b_ref[...])

  return pl.pallas_call(
      body,
      grid=_grid(n),
      in_specs=[_row_spec(f), _full_spec(wr), _full_spec(b)],
      out_specs=_row_spec(c),
      out_shape=jax.ShapeDtypeStruct((n, c), jnp.float32),
  )(hv, wr, b)


def _tc_mid(agg, inv, z, wl):
  n, h = z.shape
  c = wl.shape[1]

  def body(p_ref, i_ref, z_ref, wl_ref, y_ref, h_ref):
    hv = jnp.maximum(
        (p_ref[0] + p_ref[1]) * i_ref[...].reshape(BN, 1) + z_ref[...],
        0.0)
    h_ref[...] = hv
    y_ref[...] = jnp.dot(hv, wl_ref[...],
                         preferred_element_type=jnp.float32)

  return pl.pallas_call(
      body,
      grid=_grid(n),
      in_specs=[_part_spec(h), _vec_spec(), _row_spec(h),
                _full_spec(wl)],
      out_specs=(_row_spec(c), _row_spec(h)),
      out_shape=(jax.ShapeDtypeStruct((n, c), jnp.float32),
                 jax.ShapeDtypeStruct((n, h), jnp.float32)),
  )(agg, inv, z, wl)


def _tc_final(agg, inv, z):
  n, c = z.shape

  def body(p_ref, i_ref, z_ref, o_ref):
    out = ((p_ref[0] + p_ref[1]) * i_ref[...].reshape(BN, 1)
           + z_ref[...])
    nrm = jnp.sqrt(jnp.sum(out * out, axis=1, keepdims=True))
    out = out / jnp.maximum(nrm, 1e-12)
    m = jnp.max(out, axis=1, keepdims=True)
    o_ref[...] = out - (m + jnp.log(jnp.sum(jnp.exp(out - m), axis=1,
                                            keepdims=True)))

  return pl.pallas_call(
      body,
      grid=_grid(n),
      in_specs=[_part_spec(c), _vec_spec(), _row_spec(c)],
      out_specs=_row_spec(c),
      out_shape=jax.ShapeDtypeStruct((n, c), jnp.float32),
  )(agg, inv, z)


def kernel(x, edge_index, Wl0, Wr0, b0, Wl1, Wr1, b1, Wl2, Wr2, b2):
  n, f = x.shape
  e = edge_index.shape[1]

  cpw = -(-e // (NW * CHUNK))
  cpw = -(-cpw // 16) * 16
  e_pad = NW * cpw * CHUNK
  n_pad = -(-n // CHUNK) * CHUNK
  if e_pad > e and n_pad == n:
    n_pad += CHUNK

  src = edge_index[0]
  dst = edge_index[1]
  pad = e_pad - e
  if pad:
    pidx = np.arange(pad, dtype=np.int32)
    src = jnp.concatenate([src, jnp.asarray(pidx % n)])
    dst = jnp.concatenate([dst, jnp.asarray(n + pidx % (n_pad - n))])
  src = src.reshape(NW, cpw, CHUNK)
  dst = dst.reshape(NW, cpw, CHUNK)

  h = Wl0.shape[1]
  c_out = Wl2.shape[1]
  sc_first = _make_sc_agg(n_pad, h, cpw, True)
  sc_mid = _make_sc_agg(n_pad, h, cpw, False)
  sc_last = _make_sc_agg(n_pad, c_out, cpw, False)

  agg0, cnt = sc_first(x, src, dst)
  y1, h1, inv = _tc_first(agg0, cnt[:n_pad], cnt[n_pad:], x, Wl0, Wr0,
                          b0.reshape(1, -1), Wl1)
  agg1 = sc_mid(y1, src, dst)
  z1 = _tc_z(h1, Wr1, b1.reshape(1, -1))
  y2, h2 = _tc_mid(agg1, inv, z1, Wl2)
  agg2 = sc_last(y2, src, dst)
  z2 = _tc_z(h2, Wr2, b2.reshape(1, -1))
  return _tc_final(agg2, inv, z2)
